# Initial kernel scaffold; baseline (speedup 1.0000x reference)
import jax
import jax.numpy as jnp
from jax.experimental import pallas as pl


def kernel(i, p0, p1, p2):
    raise NotImplementedError("write your pallas kernel here")


# speedup vs baseline: 1.1231x; 1.1231x over previous
import functools

import jax
import jax.numpy as jnp
from jax import lax
from jax.experimental import pallas as pl
from jax.experimental.pallas import tpu as pltpu
from jax.experimental.pallas import tpu_sc as plsc

_N = 4096
_B = 2
_C = 1024
_ROWS = _N * _B
_NW = 32
_RPW = _ROWS // _NW
_S = 16
_NSUB = _RPW // _S
_NC = 2


@functools.partial(
    pl.kernel,
    mesh=plsc.VectorSubcoreMesh(core_axis_name="c", subcore_axis_name="s"),
    out_type=jax.ShapeDtypeStruct((_ROWS, _C), jnp.float32),
    scratch_types=[
        pltpu.VMEM((_NSUB, 3 * _S), jnp.int32),
        pltpu.VMEM((3 * _S, _C), jnp.float32),
        pltpu.VMEM((_S, _C), jnp.float32),
        pltpu.SemaphoreType.DMA,
    ],
)
def _pe_gather_sum(idx_hbm, table_hbm, out_hbm, idx_v, buf_v, out_v, sem):
    wid = lax.axis_index("s") * _NC + lax.axis_index("c")
    pltpu.sync_copy(idx_hbm.at[wid], idx_v)

    def sub(s, carry):
        pltpu.async_copy(table_hbm.at[idx_v.at[s]], buf_v, sem).wait()

        def row(j, c2):
            for c0 in range(0, _C, 16):
                a = buf_v[3 * j, pl.ds(c0, 16)]
                b = buf_v[3 * j + 1, pl.ds(c0, 16)]
                c = buf_v[3 * j + 2, pl.ds(c0, 16)]
                out_v[j, pl.ds(c0, 16)] = a + b + c
            return c2

        lax.fori_loop(0, _S, row, 0)
        row0 = wid * _RPW + s * _S
        pltpu.sync_copy(out_v, out_hbm.at[pl.ds(row0, _S)])
        return carry

    lax.fori_loop(0, _NSUB, sub, 0)


def kernel(i, p0, p1, p2):
    ii = i.reshape(_ROWS, 3).astype(jnp.int32)
    idx = ii + jnp.array([0, 16, 32], jnp.int32)
    idx = idx.reshape(_NW, _NSUB, 3 * _S)
    table = jnp.concatenate([p0, p1, p2], axis=0)
    pe = _pe_gather_sum(idx, table)
    return pe.reshape(_N, _B, _C), jnp.zeros((_N, _N, _B), dtype=bool)
